# baseline (speedup 1.0000x reference)
import dataclasses
import functools

import jax
import jax.numpy as jnp
from jax import lax
from jax.experimental import pallas as pl
from jax.experimental.pallas import tpu as pltpu
from jax.experimental.pallas import tpu_sc as plsc

N_NODES = 10000
F = 128
N_GRAPHS = 128
NC, NS = 2, 16
NW = NC * NS
CHUNK = 128
HALF_N = 5000
CORE_ROWS = 5120
DUMMY = HALF_N
N_PAD = NC * CORE_ROWS
SCHUNKS = 160
E_PAD = NS * SCHUNKS * CHUNK
NDW = 4
DCHUNKS = E_PAD // (NC * NDW * CHUNK)
HALF = CORE_ROWS
RPS = CORE_ROWS // NS

_sc_mesh = plsc.VectorSubcoreMesh(core_axis_name="c", subcore_axis_name="s")

_sc_params = pltpu.CompilerParams()
if "needs_layout_passes" in pltpu.CompilerParams.__dataclass_fields__:
    _sc_params = dataclasses.replace(_sc_params, needs_layout_passes=False)


HROWS = HALF // 8


def _sc_degree(dst_idx, zeros):
    @functools.partial(
        pl.kernel,
        out_type=jax.ShapeDtypeStruct((NC, NDW, HROWS, CHUNK), jnp.int32),
        mesh=_sc_mesh,
        compiler_params=_sc_params,
        scratch_types=[
            pltpu.VMEM((DCHUNKS, CHUNK), jnp.int32),
            pltpu.VMEM((HROWS, CHUNK), jnp.int32),
        ],
    )
    def k(dst_hbm, z_hbm, out_hbm, dst_v, hist):
        cid = lax.axis_index("c")
        sid = lax.axis_index("s")

        @pl.when(sid < NDW)
        def _():
            wid = cid * NDW + sid
            pltpu.sync_copy(dst_hbm.at[wid], dst_v)
            pltpu.sync_copy(z_hbm, hist)
            iota = lax.iota(jnp.int32, 16)

            @pl.loop(0, DCHUNKS)
            def _(j):
                @pl.loop(0, CHUNK, step=16)
                def _(kk):
                    kk = pl.multiple_of(kk, 16)
                    d = dst_v[j, pl.ds(kk, 16)]
                    hi = d >= HALF
                    local = d - jnp.where(hi, HALF, 0)
                    val = jnp.where(hi, 1 << 16, 1)
                    row = local >> 3
                    lane = ((local & 7) << 4) + iota
                    plsc.addupdate_scatter(hist, [row, lane], val)

            pltpu.sync_copy(hist, out_hbm.at[cid, sid])

    return k(dst_idx, zeros)


def _make_sc_spmm():
    @functools.partial(
        pl.kernel,
        out_type=jax.ShapeDtypeStruct((NC, CORE_ROWS, F), jnp.float32),
        mesh=_sc_mesh,
        compiler_params=_sc_params,
        scratch_types=[
            pltpu.VMEM((SCHUNKS, CHUNK), jnp.int32),
            pltpu.VMEM((SCHUNKS, CHUNK), jnp.int32),
            pltpu.VMEM((CHUNK, F), jnp.float32),
            pltpu.VMEM_SHARED((CORE_ROWS, F), jnp.float32),
            pltpu.SemaphoreType.DMA,
        ],
    )
    def k(scaled_hbm, src_hbm, dst_hbm, out_hbm, src_v, dst_v,
          rows_v, acc_sh, sem):
        cid = lax.axis_index("c")
        sid = lax.axis_index("s")
        base = cid * CORE_ROWS
        pltpu.sync_copy(scaled_hbm.at[pl.ds(base + sid * RPS, RPS)],
                        acc_sh.at[pl.ds(sid * RPS, RPS)])
        pltpu.sync_copy(src_hbm.at[sid], src_v)
        pltpu.sync_copy(dst_hbm.at[sid], dst_v)

        @pl.loop(0, SCHUNKS)
        def _(j):
            @pl.loop(0, CHUNK, step=16)
            def _(kk):
                kk = pl.multiple_of(kk, 16)
                d = dst_v[j, pl.ds(kk, 16)]
                local = d - base
                msk = (local >= 0) & (local < CORE_ROWS)
                dst_v[j, pl.ds(kk, 16)] = jnp.where(msk, local, DUMMY)

        plsc.subcore_barrier()

        @pl.loop(0, SCHUNKS)
        def _(j):
            pltpu.async_copy(scaled_hbm.at[src_v.at[j]], rows_v, sem).wait()
            pltpu.sync_copy(rows_v, acc_sh.at[dst_v.at[j]], add=True)

        plsc.subcore_barrier()
        pltpu.sync_copy(acc_sh.at[pl.ds(sid * RPS, RPS)],
                        out_hbm.at[cid, pl.ds(sid * RPS, RPS)])

    return k


_sc_spmm = _make_sc_spmm()


def _tc_dis(hists):
    def body(h_ref, o_ref):
        h = h_ref[...]
        lo = jnp.sum(h & 0xFFFF, axis=(0, 1))
        hi = jnp.sum(h >> 16, axis=(0, 1))
        for p, s in enumerate((lo, hi)):
            cnt = jnp.sum(s.reshape(HROWS, 8, 16), axis=2)
            deg = cnt.reshape(HALF, 1).astype(jnp.float32) + 1.0
            o_ref[pl.ds(p * HALF, HALF), :] = lax.rsqrt(deg)

    return pl.pallas_call(
        body, out_shape=jax.ShapeDtypeStruct((N_PAD, 1), jnp.float32))(hists)


def _tc_matmul(x_pad, w):
    def body(x_ref, w_ref, o_ref):
        o_ref[...] = jnp.dot(x_ref[...], w_ref[...],
                             preferred_element_type=jnp.float32)

    return pl.pallas_call(
        body, out_shape=jax.ShapeDtypeStruct((N_PAD, F), jnp.float32))(x_pad, w)


def _tc_prescale(y, dis):
    def body(y_ref, d_ref, o_ref):
        o_ref[...] = d_ref[...] * y_ref[...]

    return pl.pallas_call(
        body, out_shape=jax.ShapeDtypeStruct((N_PAD, F), jnp.float32))(y, dis)


def _agg_h(a_ref, s_ref, d_ref, b_ref):
    del s_ref
    a = jnp.concatenate([a_ref[0], a_ref[1]], axis=0)
    h = jnp.maximum(d_ref[...] * a + b_ref[...], 0.0)
    rows = lax.broadcasted_iota(jnp.int32, (N_PAD, 1), 0)
    return jnp.where((rows % CORE_ROWS) < HALF_N, h, 0.0)


def _tc_combine_matmul(agg, scaled, dis, b, w):
    def body(a_ref, s_ref, d_ref, b_ref, w_ref, o_ref):
        h = _agg_h(a_ref, s_ref, d_ref, b_ref)
        o_ref[...] = d_ref[...] * jnp.dot(h, w_ref[...],
                                          preferred_element_type=jnp.float32)

    return pl.pallas_call(
        body, out_shape=jax.ShapeDtypeStruct((N_PAD, F), jnp.float32))(
            agg, scaled, dis, b, w)


def _tc_finish(agg, scaled, dis, b, batch_row, wl, bl):
    def body(a_ref, s_ref, d_ref, b_ref, g_ref, wl_ref, bl_ref, o_ref):
        h = _agg_h(a_ref, s_ref, d_ref, b_ref)
        gid = lax.broadcasted_iota(jnp.int32, (N_GRAPHS, N_PAD), 0)
        pt = (gid == g_ref[...]).astype(jnp.float32)
        sums = jnp.dot(pt, h, preferred_element_type=jnp.float32)
        cnt = jnp.sum(pt, axis=1)[:, None]
        pooled = sums / jnp.maximum(cnt, 1.0)
        o_ref[...] = jnp.dot(pooled, wl_ref[...],
                             preferred_element_type=jnp.float32) + bl_ref[...]

    return pl.pallas_call(
        body, out_shape=jax.ShapeDtypeStruct((N_GRAPHS, F), jnp.float32))(
            agg, scaled, dis, b, batch_row, wl, bl)


def kernel(x, edge_index, batch, W1, b1, W2, b2, Wl, bl):
    i32 = jnp.int32
    src = edge_index[0].astype(i32)
    dst = edge_index[1].astype(i32)
    src = src + jnp.where(src >= HALF_N, CORE_ROWS - HALF_N, 0)
    dst = dst + jnp.where(dst >= HALF_N, CORE_ROWS - HALF_N, 0)
    e = src.shape[0]
    pad = jnp.full((E_PAD - e,), DUMMY, i32)
    src_flat = jnp.concatenate([src, pad])
    dst_flat = jnp.concatenate([dst, pad])
    src16 = src_flat.reshape(NS, SCHUNKS, CHUNK)
    dst16 = dst_flat.reshape(NS, SCHUNKS, CHUNK)
    dst8 = dst_flat.reshape(NC * NDW, DCHUNKS, CHUNK)
    zrow = jnp.zeros((CORE_ROWS - HALF_N, F), x.dtype)
    x_pad = jnp.concatenate([x[:HALF_N], zrow, x[HALF_N:], zrow])
    gpad = jnp.full((CORE_ROWS - HALF_N,), N_GRAPHS, i32)
    b32 = batch.astype(i32)
    batch_row = jnp.concatenate([b32[:HALF_N], gpad, b32[HALF_N:],
                                 gpad]).reshape(1, N_PAD)

    zeros = jnp.zeros((HROWS, CHUNK), i32)
    hists = _sc_degree(dst8, zeros)
    y1 = _tc_matmul(x_pad, W1)
    dis = _tc_dis(hists)
    scaled1 = _tc_prescale(y1, dis)
    agg1 = _sc_spmm(scaled1, src16, dst16)
    scaled2 = _tc_combine_matmul(agg1, scaled1, dis, b1.reshape(1, F), W2)
    agg2 = _sc_spmm(scaled2, src16, dst16)
    return _tc_finish(agg2, scaled2, dis, b2.reshape(1, F),
                      batch_row, Wl, bl.reshape(1, F))


# speedup vs baseline: 2.2965x; 2.2965x over previous
import dataclasses
import functools

import jax
import jax.numpy as jnp
from jax import lax
from jax.experimental import pallas as pl
from jax.experimental.pallas import tpu as pltpu
from jax.experimental.pallas import tpu_sc as plsc

N_NODES = 10000
F = 128
N_GRAPHS = 128
NC, NS = 2, 16
NW = NC * NS
CHUNK = 128
HALF_N = 5000
CORE_ROWS = 5120
DUMMY = HALF_N
N_PAD = NC * CORE_ROWS
SCHUNKS = 160
E_PAD = NS * SCHUNKS * CHUNK
NDW = 4
DCHUNKS = E_PAD // (NC * NDW * CHUNK)
HALF = CORE_ROWS
RPS = CORE_ROWS // NS

_sc_mesh = plsc.VectorSubcoreMesh(core_axis_name="c", subcore_axis_name="s")

_sc_params = pltpu.CompilerParams()
if "needs_layout_passes" in pltpu.CompilerParams.__dataclass_fields__:
    _sc_params = dataclasses.replace(_sc_params, needs_layout_passes=False)


HROWS = HALF // 8


def _sc_degree(dst_idx, zeros):
    @functools.partial(
        pl.kernel,
        out_type=jax.ShapeDtypeStruct((NC, NDW, HROWS, CHUNK), jnp.int32),
        mesh=_sc_mesh,
        compiler_params=_sc_params,
        scratch_types=[
            pltpu.VMEM((DCHUNKS, CHUNK), jnp.int32),
            pltpu.VMEM((HROWS, CHUNK), jnp.int32),
        ],
    )
    def k(dst_hbm, z_hbm, out_hbm, dst_v, hist):
        cid = lax.axis_index("c")
        sid = lax.axis_index("s")

        @pl.when(sid < NDW)
        def _():
            wid = cid * NDW + sid
            pltpu.sync_copy(dst_hbm.at[wid], dst_v)
            pltpu.sync_copy(z_hbm, hist)
            iota = lax.iota(jnp.int32, 16)

            @pl.loop(0, DCHUNKS)
            def _(j):
                @pl.loop(0, CHUNK, step=16)
                def _(kk):
                    kk = pl.multiple_of(kk, 16)
                    d = dst_v[j, pl.ds(kk, 16)]
                    hi = d >= HALF
                    local = d - jnp.where(hi, HALF, 0)
                    val = jnp.where(hi, 1 << 16, 1)
                    row = local >> 3
                    lane = ((local & 7) << 4) + iota
                    plsc.addupdate_scatter(hist, [row, lane], val)

            pltpu.sync_copy(hist, out_hbm.at[cid, sid])

    return k(dst_idx, zeros)


def _make_sc_spmm():
    @functools.partial(
        pl.kernel,
        out_type=jax.ShapeDtypeStruct((NC, CORE_ROWS, F), jnp.float32),
        mesh=_sc_mesh,
        compiler_params=_sc_params,
        scratch_types=[
            pltpu.VMEM((SCHUNKS, CHUNK), jnp.int32),
            pltpu.VMEM((SCHUNKS, CHUNK), jnp.int32),
            pltpu.VMEM((CHUNK, F), jnp.float32),
            pltpu.VMEM_SHARED((CORE_ROWS, F), jnp.float32),
            pltpu.SemaphoreType.DMA,
        ],
    )
    def k(scaled_hbm, src_hbm, dst_hbm, out_hbm, src_v, dst_v,
          rows_v, acc_sh, sem):
        cid = lax.axis_index("c")
        sid = lax.axis_index("s")
        base = cid * CORE_ROWS
        pltpu.sync_copy(scaled_hbm.at[pl.ds(base + sid * RPS, RPS)],
                        acc_sh.at[pl.ds(sid * RPS, RPS)])
        pltpu.sync_copy(src_hbm.at[sid], src_v)
        pltpu.sync_copy(dst_hbm.at[sid], dst_v)

        @pl.loop(0, SCHUNKS)
        def _(j):
            @pl.loop(0, CHUNK, step=16)
            def _(kk):
                kk = pl.multiple_of(kk, 16)
                d = dst_v[j, pl.ds(kk, 16)]
                local = d - base
                msk = (local >= 0) & (local < CORE_ROWS)
                dst_v[j, pl.ds(kk, 16)] = jnp.where(msk, local,
                                                    DUMMY + (d & 63))

        plsc.subcore_barrier()

        @pl.loop(0, SCHUNKS)
        def _(j):
            pltpu.async_copy(scaled_hbm.at[src_v.at[j]], rows_v, sem).wait()
            pltpu.sync_copy(rows_v, acc_sh.at[dst_v.at[j]], add=True)

        plsc.subcore_barrier()
        pltpu.sync_copy(acc_sh.at[pl.ds(sid * RPS, RPS)],
                        out_hbm.at[cid, pl.ds(sid * RPS, RPS)])

    return k


_sc_spmm = _make_sc_spmm()


def _tc_dis(hists):
    def body(h_ref, o_ref):
        h = h_ref[...]
        lo = jnp.sum(h & 0xFFFF, axis=(0, 1))
        hi = jnp.sum(h >> 16, axis=(0, 1))
        for p, s in enumerate((lo, hi)):
            cnt = jnp.sum(s.reshape(HROWS, 8, 16), axis=2)
            deg = cnt.reshape(HALF, 1).astype(jnp.float32) + 1.0
            o_ref[pl.ds(p * HALF, HALF), :] = lax.rsqrt(deg)

    return pl.pallas_call(
        body, out_shape=jax.ShapeDtypeStruct((N_PAD, 1), jnp.float32))(hists)


def _tc_matmul(x_pad, w):
    def body(x_ref, w_ref, o_ref):
        o_ref[...] = jnp.dot(x_ref[...], w_ref[...],
                             preferred_element_type=jnp.float32)

    return pl.pallas_call(
        body, out_shape=jax.ShapeDtypeStruct((N_PAD, F), jnp.float32))(x_pad, w)


def _tc_prescale(y, dis):
    def body(y_ref, d_ref, o_ref):
        o_ref[...] = d_ref[...] * y_ref[...]

    return pl.pallas_call(
        body, out_shape=jax.ShapeDtypeStruct((N_PAD, F), jnp.float32))(y, dis)


def _agg_h(a_ref, s_ref, d_ref, b_ref):
    del s_ref
    a = jnp.concatenate([a_ref[0], a_ref[1]], axis=0)
    h = jnp.maximum(d_ref[...] * a + b_ref[...], 0.0)
    rows = lax.broadcasted_iota(jnp.int32, (N_PAD, 1), 0)
    return jnp.where((rows % CORE_ROWS) < HALF_N, h, 0.0)


def _tc_combine_matmul(agg, scaled, dis, b, w):
    def body(a_ref, s_ref, d_ref, b_ref, w_ref, o_ref):
        h = _agg_h(a_ref, s_ref, d_ref, b_ref)
        o_ref[...] = d_ref[...] * jnp.dot(h, w_ref[...],
                                          preferred_element_type=jnp.float32)

    return pl.pallas_call(
        body, out_shape=jax.ShapeDtypeStruct((N_PAD, F), jnp.float32))(
            agg, scaled, dis, b, w)


def _tc_finish(agg, scaled, dis, b, batch_row, wl, bl):
    def body(a_ref, s_ref, d_ref, b_ref, g_ref, wl_ref, bl_ref, o_ref):
        h = _agg_h(a_ref, s_ref, d_ref, b_ref)
        gid = lax.broadcasted_iota(jnp.int32, (N_GRAPHS, N_PAD), 0)
        pt = (gid == g_ref[...]).astype(jnp.float32)
        sums = jnp.dot(pt, h, preferred_element_type=jnp.float32)
        cnt = jnp.sum(pt, axis=1)[:, None]
        pooled = sums / jnp.maximum(cnt, 1.0)
        o_ref[...] = jnp.dot(pooled, wl_ref[...],
                             preferred_element_type=jnp.float32) + bl_ref[...]

    return pl.pallas_call(
        body, out_shape=jax.ShapeDtypeStruct((N_GRAPHS, F), jnp.float32))(
            agg, scaled, dis, b, batch_row, wl, bl)


def kernel(x, edge_index, batch, W1, b1, W2, b2, Wl, bl):
    i32 = jnp.int32
    src = edge_index[0].astype(i32)
    dst = edge_index[1].astype(i32)
    src = src + jnp.where(src >= HALF_N, CORE_ROWS - HALF_N, 0)
    dst = dst + jnp.where(dst >= HALF_N, CORE_ROWS - HALF_N, 0)
    e = src.shape[0]
    pad = DUMMY + (jnp.arange(E_PAD - e, dtype=i32) & 63)
    src_flat = jnp.concatenate([src, pad])
    dst_flat = jnp.concatenate([dst, pad])
    src16 = src_flat.reshape(NS, SCHUNKS, CHUNK)
    dst16 = dst_flat.reshape(NS, SCHUNKS, CHUNK)
    dst8 = dst_flat.reshape(NC * NDW, DCHUNKS, CHUNK)
    zrow = jnp.zeros((CORE_ROWS - HALF_N, F), x.dtype)
    x_pad = jnp.concatenate([x[:HALF_N], zrow, x[HALF_N:], zrow])
    gpad = jnp.full((CORE_ROWS - HALF_N,), N_GRAPHS, i32)
    b32 = batch.astype(i32)
    batch_row = jnp.concatenate([b32[:HALF_N], gpad, b32[HALF_N:],
                                 gpad]).reshape(1, N_PAD)

    zeros = jnp.zeros((HROWS, CHUNK), i32)
    hists = _sc_degree(dst8, zeros)
    y1 = _tc_matmul(x_pad, W1)
    dis = _tc_dis(hists)
    scaled1 = _tc_prescale(y1, dis)
    agg1 = _sc_spmm(scaled1, src16, dst16)
    scaled2 = _tc_combine_matmul(agg1, scaled1, dis, b1.reshape(1, F), W2)
    agg2 = _sc_spmm(scaled2, src16, dst16)
    return _tc_finish(agg2, scaled2, dis, b2.reshape(1, F),
                      batch_row, Wl, bl.reshape(1, F))


# speedup vs baseline: 3.6065x; 1.5704x over previous
import dataclasses
import functools

import jax
import jax.numpy as jnp
from jax import lax
from jax.experimental import pallas as pl
from jax.experimental.pallas import tpu as pltpu
from jax.experimental.pallas import tpu_sc as plsc

N_NODES = 10000
F = 128
N_GRAPHS = 128
NC, NS = 2, 16
NW = NC * NS
CHUNK = 128
HALF_N = 5000
CORE_ROWS = 5120
DUMMY = HALF_N
N_PAD = NC * CORE_ROWS
SCHUNKS = 160
E_PAD = NS * SCHUNKS * CHUNK
NDW = 4
DCHUNKS = E_PAD // (NC * NDW * CHUNK)
HALF = CORE_ROWS
RPS = CORE_ROWS // NS

_sc_mesh = plsc.VectorSubcoreMesh(core_axis_name="c", subcore_axis_name="s")

_sc_params = pltpu.CompilerParams()
if "needs_layout_passes" in pltpu.CompilerParams.__dataclass_fields__:
    _sc_params = dataclasses.replace(_sc_params, needs_layout_passes=False)


HROWS = HALF // 8


def _sc_degree(dst_idx, zeros):
    @functools.partial(
        pl.kernel,
        out_type=jax.ShapeDtypeStruct((NC, NDW, HROWS, CHUNK), jnp.int32),
        mesh=_sc_mesh,
        compiler_params=_sc_params,
        scratch_types=[
            pltpu.VMEM((DCHUNKS, CHUNK), jnp.int32),
            pltpu.VMEM((HROWS, CHUNK), jnp.int32),
        ],
    )
    def k(dst_hbm, z_hbm, out_hbm, dst_v, hist):
        cid = lax.axis_index("c")
        sid = lax.axis_index("s")

        @pl.when(sid < NDW)
        def _():
            wid = cid * NDW + sid
            pltpu.sync_copy(dst_hbm.at[wid], dst_v)
            pltpu.sync_copy(z_hbm, hist)
            iota = lax.iota(jnp.int32, 16)

            @pl.loop(0, DCHUNKS)
            def _(j):
                @pl.loop(0, CHUNK, step=16)
                def _(kk):
                    kk = pl.multiple_of(kk, 16)
                    d = dst_v[j, pl.ds(kk, 16)]
                    hi = d >= HALF
                    local = d - jnp.where(hi, HALF, 0)
                    val = jnp.where(hi, 1 << 16, 1)
                    row = local >> 3
                    lane = ((local & 7) << 4) + iota
                    plsc.addupdate_scatter(hist, [row, lane], val)

            pltpu.sync_copy(hist, out_hbm.at[cid, sid])

    return k(dst_idx, zeros)


def _make_sc_spmm():
    @functools.partial(
        pl.kernel,
        out_type=jax.ShapeDtypeStruct((NC, CORE_ROWS, F), jnp.float32),
        mesh=_sc_mesh,
        compiler_params=_sc_params,
        scratch_types=[
            pltpu.VMEM((SCHUNKS, CHUNK), jnp.int32),
            pltpu.VMEM((SCHUNKS, CHUNK), jnp.int32),
            pltpu.VMEM((CHUNK, F), jnp.float32),
            pltpu.VMEM((CHUNK, F), jnp.float32),
            pltpu.VMEM_SHARED((CORE_ROWS, F), jnp.float32),
            pltpu.SemaphoreType.DMA,
            pltpu.SemaphoreType.DMA,
        ],
    )
    def k(scaled_hbm, src_hbm, dst_hbm, out_hbm, src_v, dst_v,
          rows_a, rows_b, acc_sh, sem_a, sem_b):
        cid = lax.axis_index("c")
        sid = lax.axis_index("s")
        base = cid * CORE_ROWS
        pltpu.sync_copy(scaled_hbm.at[pl.ds(base + sid * RPS, RPS)],
                        acc_sh.at[pl.ds(sid * RPS, RPS)])
        pltpu.sync_copy(src_hbm.at[sid], src_v)
        pltpu.sync_copy(dst_hbm.at[sid], dst_v)

        @pl.loop(0, SCHUNKS)
        def _(j):
            @pl.loop(0, CHUNK, step=16)
            def _(kk):
                kk = pl.multiple_of(kk, 16)
                d = dst_v[j, pl.ds(kk, 16)]
                local = d - base
                msk = (local >= 0) & (local < CORE_ROWS)
                dst_v[j, pl.ds(kk, 16)] = jnp.where(msk, local,
                                                    DUMMY + (d & 63))

        plsc.subcore_barrier()

        def wait_gather(rows, sem):
            pltpu.make_async_copy(scaled_hbm.at[pl.ds(0, CHUNK)], rows,
                                  sem).wait()

        pltpu.async_copy(scaled_hbm.at[src_v.at[0]], rows_a, sem_a)
        pltpu.async_copy(scaled_hbm.at[src_v.at[1]], rows_b, sem_b)

        @pl.loop(0, SCHUNKS, step=2)
        def _(j):
            for (c, r, sem) in ((j, rows_a, sem_a), (j + 1, rows_b, sem_b)):
                wait_gather(r, sem)
                pltpu.sync_copy(r, acc_sh.at[dst_v.at[c]], add=True)

                @pl.when(c + 2 < SCHUNKS)
                def _():
                    pltpu.async_copy(scaled_hbm.at[src_v.at[c + 2]], r, sem)

        plsc.subcore_barrier()
        pltpu.sync_copy(acc_sh.at[pl.ds(sid * RPS, RPS)],
                        out_hbm.at[cid, pl.ds(sid * RPS, RPS)])

    return k


_sc_spmm = _make_sc_spmm()


def _tc_dis(hists):
    def body(h_ref, o_ref):
        h = h_ref[...]
        lo = jnp.sum(h & 0xFFFF, axis=(0, 1))
        hi = jnp.sum(h >> 16, axis=(0, 1))
        for p, s in enumerate((lo, hi)):
            cnt = jnp.sum(s.reshape(HROWS, 8, 16), axis=2)
            deg = cnt.reshape(HALF, 1).astype(jnp.float32) + 1.0
            o_ref[pl.ds(p * HALF, HALF), :] = lax.rsqrt(deg)

    return pl.pallas_call(
        body, out_shape=jax.ShapeDtypeStruct((N_PAD, 1), jnp.float32))(hists)


def _tc_matmul(x_pad, w):
    def body(x_ref, w_ref, o_ref):
        o_ref[...] = jnp.dot(x_ref[...], w_ref[...],
                             preferred_element_type=jnp.float32)

    return pl.pallas_call(
        body, out_shape=jax.ShapeDtypeStruct((N_PAD, F), jnp.float32))(x_pad, w)


def _tc_prescale(y, dis):
    def body(y_ref, d_ref, o_ref):
        o_ref[...] = d_ref[...] * y_ref[...]

    return pl.pallas_call(
        body, out_shape=jax.ShapeDtypeStruct((N_PAD, F), jnp.float32))(y, dis)


def _agg_h(a_ref, s_ref, d_ref, b_ref):
    del s_ref
    a = jnp.concatenate([a_ref[0], a_ref[1]], axis=0)
    h = jnp.maximum(d_ref[...] * a + b_ref[...], 0.0)
    rows = lax.broadcasted_iota(jnp.int32, (N_PAD, 1), 0)
    return jnp.where((rows % CORE_ROWS) < HALF_N, h, 0.0)


def _tc_combine_matmul(agg, scaled, dis, b, w):
    def body(a_ref, s_ref, d_ref, b_ref, w_ref, o_ref):
        h = _agg_h(a_ref, s_ref, d_ref, b_ref)
        o_ref[...] = d_ref[...] * jnp.dot(h, w_ref[...],
                                          preferred_element_type=jnp.float32)

    return pl.pallas_call(
        body, out_shape=jax.ShapeDtypeStruct((N_PAD, F), jnp.float32))(
            agg, scaled, dis, b, w)


def _tc_finish(agg, scaled, dis, b, batch_row, wl, bl):
    def body(a_ref, s_ref, d_ref, b_ref, g_ref, wl_ref, bl_ref, o_ref):
        h = _agg_h(a_ref, s_ref, d_ref, b_ref)
        gid = lax.broadcasted_iota(jnp.int32, (N_GRAPHS, N_PAD), 0)
        pt = (gid == g_ref[...]).astype(jnp.float32)
        sums = jnp.dot(pt, h, preferred_element_type=jnp.float32)
        cnt = jnp.sum(pt, axis=1)[:, None]
        pooled = sums / jnp.maximum(cnt, 1.0)
        o_ref[...] = jnp.dot(pooled, wl_ref[...],
                             preferred_element_type=jnp.float32) + bl_ref[...]

    return pl.pallas_call(
        body, out_shape=jax.ShapeDtypeStruct((N_GRAPHS, F), jnp.float32))(
            agg, scaled, dis, b, batch_row, wl, bl)


def kernel(x, edge_index, batch, W1, b1, W2, b2, Wl, bl):
    i32 = jnp.int32
    src = edge_index[0].astype(i32)
    dst = edge_index[1].astype(i32)
    src = src + jnp.where(src >= HALF_N, CORE_ROWS - HALF_N, 0)
    dst = dst + jnp.where(dst >= HALF_N, CORE_ROWS - HALF_N, 0)
    e = src.shape[0]
    pad = DUMMY + (jnp.arange(E_PAD - e, dtype=i32) & 63)
    src_flat = jnp.concatenate([src, pad])
    dst_flat = jnp.concatenate([dst, pad])
    src16 = src_flat.reshape(NS, SCHUNKS, CHUNK)
    dst16 = dst_flat.reshape(NS, SCHUNKS, CHUNK)
    dst8 = dst_flat.reshape(NC * NDW, DCHUNKS, CHUNK)
    zrow = jnp.zeros((CORE_ROWS - HALF_N, F), x.dtype)
    x_pad = jnp.concatenate([x[:HALF_N], zrow, x[HALF_N:], zrow])
    gpad = jnp.full((CORE_ROWS - HALF_N,), N_GRAPHS, i32)
    b32 = batch.astype(i32)
    batch_row = jnp.concatenate([b32[:HALF_N], gpad, b32[HALF_N:],
                                 gpad]).reshape(1, N_PAD)

    zeros = jnp.zeros((HROWS, CHUNK), i32)
    hists = _sc_degree(dst8, zeros)
    y1 = _tc_matmul(x_pad, W1)
    dis = _tc_dis(hists)
    scaled1 = _tc_prescale(y1, dis)
    agg1 = _sc_spmm(scaled1, src16, dst16)
    scaled2 = _tc_combine_matmul(agg1, scaled1, dis, b1.reshape(1, F), W2)
    agg2 = _sc_spmm(scaled2, src16, dst16)
    return _tc_finish(agg2, scaled2, dis, b2.reshape(1, F),
                      batch_row, Wl, bl.reshape(1, F))
